# baseline (speedup 1.0000x reference)
import functools

import jax
import jax.numpy as jnp
from jax import lax
from jax.experimental import pallas as pl
from jax.experimental.pallas import tpu as pltpu

NUM_EMBEDDINGS = 8192
EMBEDDING_DIM = 256
TOKEN_TILE = 256
K_BLOCK = 2048


def _argmin_body(z_ref, e_ref, z2h_ref, e2h_ref, out_ref):
    z = z_ref[...]
    z2h = z2h_ref[...]
    n_blocks = NUM_EMBEDDINGS // K_BLOCK
    m_run = None
    i_run = None
    for j in range(n_blocks):
        ej = e_ref[pl.ds(j * K_BLOCK, K_BLOCK), :]
        e2j = e2h_ref[:, pl.ds(j * K_BLOCK, K_BLOCK)]
        mm = lax.dot_general(
            z, ej, (((1,), (1,)), ((), ())),
            preferred_element_type=jnp.float32,
        )
        d = (z2h + e2j) - mm
        bm = jnp.min(d, axis=1, keepdims=True)
        iota = lax.broadcasted_iota(
            jnp.int32, (1, K_BLOCK), 1).astype(jnp.float32) + (j * K_BLOCK)
        bi = jnp.min(
            jnp.where(d == bm, iota, jnp.float32(NUM_EMBEDDINGS)),
            axis=1, keepdims=True)
        if m_run is None:
            m_run, i_run = bm, bi
        else:
            upd = bm < m_run
            i_run = jnp.where(upd, bi, i_run)
            m_run = jnp.minimum(bm, m_run)
    out_ref[...] = i_run.astype(jnp.int32)


@functools.partial(jax.jit, static_argnames=())
def _encode(z_flat, embedding_weight, z2h, e2h):
    n_tok = z_flat.shape[0]
    grid = (n_tok // TOKEN_TILE,)
    return pl.pallas_call(
        _argmin_body,
        grid=grid,
        in_specs=[
            pl.BlockSpec((TOKEN_TILE, EMBEDDING_DIM), lambda i: (i, 0)),
            pl.BlockSpec((NUM_EMBEDDINGS, EMBEDDING_DIM), lambda i: (0, 0)),
            pl.BlockSpec((TOKEN_TILE, 1), lambda i: (i, 0)),
            pl.BlockSpec((1, NUM_EMBEDDINGS), lambda i: (0, 0)),
        ],
        out_specs=pl.BlockSpec((TOKEN_TILE, 1), lambda i: (i, 0)),
        out_shape=jax.ShapeDtypeStruct((n_tok, 1), jnp.int32),
    )(z_flat, embedding_weight, z2h, e2h)


def kernel(z_e, embedding_weight):
    B, C, H, W = z_e.shape
    z_flat = jnp.transpose(z_e.reshape(B, C, H * W), (0, 2, 1))
    z2 = jnp.sum(z_flat ** 2, axis=2, keepdims=True)
    e2 = jnp.sum(embedding_weight ** 2, axis=1)
    idx = _encode(
        z_flat.reshape(B * H * W, C),
        embedding_weight,
        (z2 * 0.5).reshape(B * H * W, 1),
        (e2 * 0.5).reshape(1, NUM_EMBEDDINGS),
    )
    encoding_indices = idx.reshape(B, H * W)
    quantized = jnp.take(embedding_weight, encoding_indices, axis=0)
    quantized = jnp.transpose(quantized, (0, 2, 1)).reshape(B, C, H, W)
    return (quantized, encoding_indices)
